# baseline (speedup 1.0000x reference)
import jax
import jax.numpy as jnp
from jax.experimental import pallas as pl
from jax.experimental.pallas import tpu as pltpu

RSTRIP = 64


def _scatter_kernel(tok_ref, wphi_ref, out_ref):
    n_tok = tok_ref.shape[0]
    rows, cols = out_ref.shape
    vids = (jax.lax.broadcasted_iota(jnp.int32, (rows, cols), 0) * cols
            + jax.lax.broadcasted_iota(jnp.int32, (rows, cols), 1))
    acc = jnp.zeros((rows, cols), jnp.float32)
    for n in range(n_tok):
        acc = acc + jnp.where(vids == tok_ref[n], wphi_ref[n], 0.0)
    out_ref[:, :] = acc


NSEM = 8
NSRC = 1


def _bcast_kernel(pen_ref, out_ref, scratch, sems):
    B = out_ref.shape[0]
    for s in range(NSRC):
        scratch[s, :, :] = jnp.broadcast_to(pen_ref[:, :], scratch.shape[1:])
    nstrips = B // RSTRIP
    for i in range(nstrips):
        pltpu.make_async_copy(
            scratch.at[i % NSRC],
            out_ref.at[pl.ds(i * RSTRIP, RSTRIP), :],
            sems.at[i % NSEM],
        ).start()
    for i in range(nstrips):
        pltpu.make_async_copy(
            scratch.at[i % NSRC],
            out_ref.at[pl.ds(i * RSTRIP, RSTRIP), :],
            sems.at[i % NSEM],
        ).wait()


def kernel(level, candidate_logits, tokens, phis, layer_weights):
    B, V = candidate_logits.shape
    pool_tokens = tokens[:, level]
    w = jnp.cumsum(layer_weights)[level]
    wphi = phis * w

    rows = 8
    cols = V // rows
    pen8 = pl.pallas_call(
        _scatter_kernel,
        grid_spec=pltpu.PrefetchScalarGridSpec(
            num_scalar_prefetch=2,
            grid=(1,),
            in_specs=[],
            out_specs=pl.BlockSpec((rows, cols), lambda i, *_: (0, 0)),
        ),
        out_shape=jax.ShapeDtypeStruct((rows, cols), jnp.float32),
    )(pool_tokens, wphi)
    pen = pen8.reshape(1, V)

    out = pl.pallas_call(
        _bcast_kernel,
        in_specs=[pl.BlockSpec(memory_space=pltpu.MemorySpace.VMEM)],
        out_specs=pl.BlockSpec(memory_space=pltpu.MemorySpace.HBM),
        out_shape=jax.ShapeDtypeStruct((B, V), jnp.float32),
        scratch_shapes=[
            pltpu.VMEM((NSRC, RSTRIP, V), jnp.float32),
            pltpu.SemaphoreType.DMA((NSEM,)),
        ],
    )(pen)
    return out


# speedup vs baseline: 2.6158x; 2.6158x over previous
import jax
import jax.numpy as jnp
from jax.experimental import pallas as pl
from jax.experimental.pallas import tpu as pltpu

VBLK = 2048


def _scatter_kernel(tok_ref, wphi_ref, out_ref):
    n_tok = tok_ref.shape[0]
    rows, cols = out_ref.shape
    vids = (jax.lax.broadcasted_iota(jnp.int32, (rows, cols), 0) * cols
            + jax.lax.broadcasted_iota(jnp.int32, (rows, cols), 1))
    acc = jnp.zeros((rows, cols), jnp.float32)
    for n in range(n_tok):
        acc = acc + jnp.where(vids == tok_ref[n], wphi_ref[n], 0.0)
    out_ref[:, :] = acc


def _bcast_kernel(pen_ref, out_ref):
    out_ref[:, :] = jnp.broadcast_to(pen_ref[:, :], out_ref.shape)


def kernel(level, candidate_logits, tokens, phis, layer_weights):
    B, V = candidate_logits.shape
    pool_tokens = tokens[:, level]
    w = jnp.cumsum(layer_weights)[level]
    wphi = phis * w

    rows = 8
    cols = V // rows
    pen8 = pl.pallas_call(
        _scatter_kernel,
        grid_spec=pltpu.PrefetchScalarGridSpec(
            num_scalar_prefetch=2,
            grid=(1,),
            in_specs=[],
            out_specs=pl.BlockSpec((rows, cols), lambda i, *_: (0, 0)),
        ),
        out_shape=jax.ShapeDtypeStruct((rows, cols), jnp.float32),
    )(pool_tokens, wphi)
    pen_col = pen8.reshape(V, 1)

    out_vb = pl.pallas_call(
        _bcast_kernel,
        grid=(pl.cdiv(V, VBLK),),
        in_specs=[pl.BlockSpec((VBLK, 1), lambda v: (v, 0))],
        out_specs=pl.BlockSpec((VBLK, B), lambda v: (v, 0)),
        out_shape=jax.ShapeDtypeStruct((V, B), jnp.float32),
    )(pen_col)
    return out_vb.T


# speedup vs baseline: 3.8272x; 1.4631x over previous
import jax
import jax.numpy as jnp
from jax.experimental import pallas as pl
from jax.experimental.pallas import tpu as pltpu

VBLK = 2048


def _fused_kernel(tok_ref, wphi_ref, out_ref):
    n_tok = tok_ref.shape[0]
    vb, b = out_ref.shape
    base = pl.program_id(0) * vb
    out_ref[:, :] = jnp.zeros((vb, b), jnp.float32)
    for n in range(n_tok):
        local = tok_ref[n] - base

        @pl.when(jnp.logical_and(local >= 0, local < vb))
        def _():
            out_ref[pl.ds(local, 1), :] = (
                out_ref[pl.ds(local, 1), :] + wphi_ref[n]
            )


def kernel(level, candidate_logits, tokens, phis, layer_weights):
    B, V = candidate_logits.shape
    pool_tokens = tokens[:, level]
    w = jnp.cumsum(layer_weights)[level]
    wphi = phis * w

    out_vb = pl.pallas_call(
        _fused_kernel,
        grid_spec=pltpu.PrefetchScalarGridSpec(
            num_scalar_prefetch=2,
            grid=(pl.cdiv(V, VBLK),),
            in_specs=[],
            out_specs=pl.BlockSpec((VBLK, B), lambda v, *_: (v, 0)),
        ),
        out_shape=jax.ShapeDtypeStruct((V, B), jnp.float32),
    )(pool_tokens, wphi)
    return out_vb.T
